# baseline (speedup 1.0000x reference)
import functools

import jax
import jax.numpy as jnp
from jax import lax
from jax.experimental import pallas as pl
from jax.experimental.pallas import tpu as pltpu
from jax.experimental.pallas import tpu_sc as plsc

N = 10000
E = 320000
D = 128
NC = 2
NS = 16
CHUNK = 128
NBUF = 3


def _make_sc_segment_sum(n_nodes, n_edges, interpret=False):
    chunks_per_core = n_edges // CHUNK // NC
    q, r = divmod(chunks_per_core, NS)
    stripe = (n_nodes // NS) // 8 * 8
    last_stripe = n_nodes - stripe * (NS - 1)

    mesh = plsc.VectorSubcoreMesh(core_axis_name="c", subcore_axis_name="s",
                                  num_cores=NC, num_subcores=NS)

    @functools.partial(
        pl.kernel,
        out_type=jax.ShapeDtypeStruct((2 * n_nodes, D), jnp.float32),
        mesh=mesh,
        scratch_types=[
            pltpu.VMEM_SHARED((n_nodes, D), jnp.float32),
            pltpu.VMEM((NBUF, CHUNK), jnp.int32),
            pltpu.VMEM((2 * NBUF, CHUNK), jnp.int32),
            pltpu.VMEM((NBUF, CHUNK, D), jnp.float32),
            pltpu.SemaphoreType.DMA((NBUF,)),
            pltpu.SemaphoreType.DMA((NBUF,)),
            pltpu.SemaphoreType.DMA((2 * NBUF,)),
            pltpu.SemaphoreType.DMA((NBUF,)),
        ],
        interpret=interpret,
    )
    def sc_seg(table, src, dst, zeros, out, agg_sh, src_i, dst_i, rows,
               gsems, isems, jsems, ssems):
        c = lax.axis_index("c")
        s = lax.axis_index("s")
        n = q + jnp.where(s < r, 1, 0)
        base = c * chunks_per_core + s * q + jnp.minimum(s, r)

        def idx_copy(j, rr, r6):
            return (
                pltpu.make_async_copy(
                    src.at[pl.ds((base + j) * CHUNK, CHUNK)], src_i.at[rr],
                    isems.at[rr]),
                pltpu.make_async_copy(
                    dst.at[pl.ds((base + j) * CHUNK, CHUNK)], dst_i.at[r6],
                    jsems.at[r6]),
            )

        def gather_copy(rr):
            return pltpu.make_async_copy(table.at[src_i.at[rr]],
                                         rows.at[rr], gsems.at[rr])

        def scatter_start(rr, r6):
            pltpu.async_copy(rows.at[rr], agg_sh.at[dst_i.at[r6]],
                             ssems.at[rr], add=True)

        def scatter_wait(rr):
            pltpu.make_async_copy(rows.at[rr], agg_sh.at[dst_i.at[0]],
                                  ssems.at[rr]).wait()

        for k in range(3):
            @pl.when(k < n)
            def _(k=k):
                for d in idx_copy(k, k % NBUF, k % (2 * NBUF)):
                    d.start()

        @pl.when(s < NS - 1)
        def _():
            pltpu.sync_copy(zeros.at[pl.ds(0, stripe)],
                            agg_sh.at[pl.ds(s * stripe, stripe)])

        @pl.when(s == NS - 1)
        def _():
            pltpu.sync_copy(zeros,
                            agg_sh.at[pl.ds((NS - 1) * stripe, last_stripe)])

        plsc.subcore_barrier()

        for d in idx_copy(0, 0, 0):
            d.wait()
        gather_copy(0).start()

        @pl.when(n > 1)
        def _():
            for d in idx_copy(1, 1, 1):
                d.wait()
            gather_copy(1).start()

        def body(j, carry):
            rr = lax.rem(j, NBUF)
            r6 = lax.rem(j, 2 * NBUF)
            gather_copy(rr).wait()
            scatter_start(rr, r6)

            @pl.when(j + 2 < n)
            def _():
                r2 = lax.rem(j + 2, NBUF)

                @pl.when(j >= 1)
                def _():
                    scatter_wait(r2)

                for d in idx_copy(j + 2, r2, lax.rem(j + 2, 2 * NBUF)):
                    d.wait()
                gather_copy(r2).start()

            @pl.when(j + 3 < n)
            def _():
                for d in idx_copy(j + 3, rr, lax.rem(j + 3, 2 * NBUF)):
                    d.start()

            return carry

        lax.fori_loop(0, n, body, jnp.int32(0))

        @pl.when(n >= 3)
        def _():
            scatter_wait(lax.rem(n - 3, NBUF))

        @pl.when(n >= 2)
        def _():
            scatter_wait(lax.rem(n - 2, NBUF))

        scatter_wait(lax.rem(n - 1, NBUF))
        plsc.subcore_barrier()

        @pl.when(s < NS - 1)
        def _():
            pltpu.sync_copy(agg_sh.at[pl.ds(s * stripe, stripe)],
                            out.at[pl.ds(c * n_nodes + s * stripe, stripe)])

        @pl.when(s == NS - 1)
        def _():
            pltpu.sync_copy(
                agg_sh.at[pl.ds((NS - 1) * stripe, last_stripe)],
                out.at[pl.ds(c * n_nodes + (NS - 1) * stripe, last_stripe)])

    return sc_seg


def _make_tc_linear(n_nodes, bn, last_layer, interpret=False):
    nb = n_nodes // bn

    def body(lo_ref, hi_ref, w_ref, b_ref, o_ref):
        a = lo_ref[...] + hi_ref[...]
        h = jnp.dot(a, w_ref[...], preferred_element_type=jnp.float32) \
            + b_ref[...]
        if last_layer:
            m = jnp.max(h, axis=-1, keepdims=True)
            e = jnp.exp(h - m)
            lse = jnp.log(jnp.sum(e, axis=-1, keepdims=True)) + m
            o_ref[...] = h - lse
        else:
            o_ref[...] = jnp.maximum(h, 0.0)

    return pl.pallas_call(
        body,
        grid=(nb,),
        in_specs=[
            pl.BlockSpec((bn, D), lambda i: (i, 0)),
            pl.BlockSpec((bn, D), lambda i: (i + nb, 0)),
            pl.BlockSpec((D, D), lambda i: (0, 0)),
            pl.BlockSpec((1, D), lambda i: (0, 0)),
        ],
        out_specs=pl.BlockSpec((bn, D), lambda i: (i, 0)),
        out_shape=jax.ShapeDtypeStruct((n_nodes, D), jnp.float32),
        interpret=interpret,
    )


def _gnn_forward(x, edge_index, weights, n_nodes, n_edges, bn=10000,
                 interpret=False):
    (W1, b1, W2, b2, W3, b3) = weights
    src = edge_index[0]
    dst = edge_index[1]
    stripe = (n_nodes // NS) // 8 * 8
    zeros = jnp.zeros((n_nodes - stripe * (NS - 1), D), jnp.float32)

    sc_seg = _make_sc_segment_sum(n_nodes, n_edges, interpret=interpret)
    tc_mid = _make_tc_linear(n_nodes, bn, last_layer=False,
                             interpret=interpret)
    tc_last = _make_tc_linear(n_nodes, bn, last_layer=True,
                              interpret=interpret)

    b1r = b1.reshape(1, D)
    b2r = b2.reshape(1, D)
    b3r = b3.reshape(1, D)

    agg = sc_seg(x, src, dst, zeros)
    h = tc_mid(agg, agg, W1, b1r)
    agg = sc_seg(h, src, dst, zeros)
    h = tc_mid(agg, agg, W2, b2r)
    agg = sc_seg(h, src, dst, zeros)
    return tc_last(agg, agg, W3, b3r)


def kernel(x, edge_index, W1, b1, W2, b2, W3, b3):
    return _gnn_forward(x, edge_index, (W1, b1, W2, b2, W3, b3), N, E)


# speedup vs baseline: 1.0113x; 1.0113x over previous
import functools

import jax
import jax.numpy as jnp
from jax import lax
from jax.experimental import pallas as pl
from jax.experimental.pallas import tpu as pltpu
from jax.experimental.pallas import tpu_sc as plsc

N = 10000
E = 320000
D = 128
NC = 2
NS = 16
CHUNK = 128
NBUF = 3


def _make_sc_segment_sum(n_nodes, n_edges, interpret=False):
    chunks_per_core = n_edges // CHUNK // NC
    q, r = divmod(chunks_per_core, NS)
    stripe = (n_nodes // NS) // 8 * 8
    last_stripe = n_nodes - stripe * (NS - 1)

    mesh = plsc.VectorSubcoreMesh(core_axis_name="c", subcore_axis_name="s",
                                  num_cores=NC, num_subcores=NS)

    @functools.partial(
        pl.kernel,
        out_type=jax.ShapeDtypeStruct((2 * n_nodes, D), jnp.float32),
        mesh=mesh,
        scratch_types=[
            pltpu.VMEM_SHARED((n_nodes, D), jnp.float32),
            pltpu.VMEM((NBUF, CHUNK), jnp.int32),
            pltpu.VMEM((2 * NBUF, CHUNK), jnp.int32),
            pltpu.VMEM((NBUF, CHUNK, D), jnp.float32),
            pltpu.SemaphoreType.DMA((NBUF,)),
            pltpu.SemaphoreType.DMA((NBUF,)),
            pltpu.SemaphoreType.DMA((2 * NBUF,)),
            pltpu.SemaphoreType.DMA((NBUF,)),
        ],
        interpret=interpret,
    )
    def sc_seg(table, src, dst, zeros, out, agg_sh, src_i, dst_i, rows,
               gsems, isems, jsems, ssems):
        c = lax.axis_index("c")
        s = lax.axis_index("s")
        n = q + jnp.where(s < r, 1, 0)
        base = c * chunks_per_core + s * q + jnp.minimum(s, r)

        def idx_copy(j, rr, r6):
            return (
                pltpu.make_async_copy(
                    src.at[pl.ds((base + j) * CHUNK, CHUNK)], src_i.at[rr],
                    isems.at[rr]),
                pltpu.make_async_copy(
                    dst.at[pl.ds((base + j) * CHUNK, CHUNK)], dst_i.at[r6],
                    jsems.at[r6]),
            )

        def gather_copy(rr):
            return pltpu.make_async_copy(table.at[src_i.at[rr]],
                                         rows.at[rr], gsems.at[rr])

        def scatter_start(rr, r6):
            pltpu.async_copy(rows.at[rr], agg_sh.at[dst_i.at[r6]],
                             ssems.at[rr], add=True)

        def scatter_wait(rr):
            pltpu.make_async_copy(rows.at[rr], agg_sh.at[dst_i.at[0]],
                                  ssems.at[rr]).wait()

        for k in range(3):
            @pl.when(k < n)
            def _(k=k):
                for d in idx_copy(k, k % NBUF, k % (2 * NBUF)):
                    d.start()

        @pl.when(s < NS - 1)
        def _():
            pltpu.sync_copy(zeros.at[pl.ds(0, stripe)],
                            agg_sh.at[pl.ds(s * stripe, stripe)])

        @pl.when(s == NS - 1)
        def _():
            pltpu.sync_copy(zeros,
                            agg_sh.at[pl.ds((NS - 1) * stripe, last_stripe)])

        plsc.subcore_barrier()

        for d in idx_copy(0, 0, 0):
            d.wait()
        gather_copy(0).start()

        @pl.when(n > 1)
        def _():
            for d in idx_copy(1, 1, 1):
                d.wait()
            gather_copy(1).start()

        def body(j, carry):
            rr = lax.rem(j, NBUF)
            r6 = lax.rem(j, 2 * NBUF)
            gather_copy(rr).wait()
            scatter_start(rr, r6)

            @pl.when(j + 2 < n)
            def _():
                r2 = lax.rem(j + 2, NBUF)

                @pl.when(j >= 1)
                def _():
                    scatter_wait(r2)

                for d in idx_copy(j + 2, r2, lax.rem(j + 2, 2 * NBUF)):
                    d.wait()
                gather_copy(r2).start()

            @pl.when(j + 3 < n)
            def _():
                for d in idx_copy(j + 3, rr, lax.rem(j + 3, 2 * NBUF)):
                    d.start()

            return carry

        lax.fori_loop(0, n, body, jnp.int32(0))

        @pl.when(n >= 3)
        def _():
            scatter_wait(lax.rem(n - 3, NBUF))

        @pl.when(n >= 2)
        def _():
            scatter_wait(lax.rem(n - 2, NBUF))

        scatter_wait(lax.rem(n - 1, NBUF))
        plsc.subcore_barrier()

        @pl.when(s < NS - 1)
        def _():
            pltpu.sync_copy(agg_sh.at[pl.ds(s * stripe, stripe)],
                            out.at[pl.ds(c * n_nodes + s * stripe, stripe)])

        @pl.when(s == NS - 1)
        def _():
            pltpu.sync_copy(
                agg_sh.at[pl.ds((NS - 1) * stripe, last_stripe)],
                out.at[pl.ds(c * n_nodes + (NS - 1) * stripe, last_stripe)])

    return sc_seg


def _make_tc_linear(n_nodes, bn, last_layer, interpret=False):
    nb = n_nodes // bn

    def body(lo_ref, hi_ref, w_ref, b_ref, o_ref):
        a = lo_ref[...] + hi_ref[...]
        h = jnp.dot(a, w_ref[...], preferred_element_type=jnp.float32) \
            + b_ref[...]
        if last_layer:
            m = jnp.max(h, axis=-1, keepdims=True)
            e = jnp.exp(h - m)
            lse = jnp.log(jnp.sum(e, axis=-1, keepdims=True)) + m
            o_ref[...] = h - lse
        else:
            o_ref[...] = jnp.maximum(h, 0.0)

    return pl.pallas_call(
        body,
        grid=(nb,),
        in_specs=[
            pl.BlockSpec((bn, D), lambda i: (i, 0)),
            pl.BlockSpec((bn, D), lambda i: (i + nb, 0)),
            pl.BlockSpec((D, D), lambda i: (0, 0)),
            pl.BlockSpec((1, D), lambda i: (0, 0)),
        ],
        out_specs=pl.BlockSpec((bn, D), lambda i: (i, 0)),
        out_shape=jax.ShapeDtypeStruct((n_nodes, D), jnp.float32),
        interpret=interpret,
    )


def _gnn_forward(x, edge_index, weights, n_nodes, n_edges, bn=5000,
                 interpret=False):
    (W1, b1, W2, b2, W3, b3) = weights
    src = edge_index[0]
    dst = edge_index[1]
    stripe = (n_nodes // NS) // 8 * 8
    zeros = jnp.zeros((n_nodes - stripe * (NS - 1), D), jnp.float32)

    sc_seg = _make_sc_segment_sum(n_nodes, n_edges, interpret=interpret)
    tc_mid = _make_tc_linear(n_nodes, bn, last_layer=False,
                             interpret=interpret)
    tc_last = _make_tc_linear(n_nodes, bn, last_layer=True,
                              interpret=interpret)

    b1r = b1.reshape(1, D)
    b2r = b2.reshape(1, D)
    b3r = b3.reshape(1, D)

    agg = sc_seg(x, src, dst, zeros)
    h = tc_mid(agg, agg, W1, b1r)
    agg = sc_seg(h, src, dst, zeros)
    h = tc_mid(agg, agg, W2, b2r)
    agg = sc_seg(h, src, dst, zeros)
    return tc_last(agg, agg, W3, b3r)


def kernel(x, edge_index, W1, b1, W2, b2, W3, b3):
    return _gnn_forward(x, edge_index, (W1, b1, W2, b2, W3, b3), N, E)
